# baseline (speedup 1.0000x reference)
import functools

import jax
import jax.numpy as jnp
from jax import lax
from jax.experimental import pallas as pl
from jax.experimental.pallas import tpu as pltpu
from jax.experimental.pallas import tpu_sc as plsc

BATCH = 4096
WORDS = 20
EMBED = 128
LANES = 16
NC, NS = 2, 16
NW = NC * NS
OBJ_PER_W = BATCH // NW
CHUNK_OBJ = 16
CHUNK_IDX = CHUNK_OBJ * WORDS
IDX_GRP = 80
GRPS = CHUNK_IDX // IDX_GRP
N_CHUNKS = OBJ_PER_W // CHUNK_OBJ
IDX_PER_W = OBJ_PER_W * WORDS
ROW_BYTES = EMBED * 4

_mesh = plsc.VectorSubcoreMesh(
    core_axis_name="c", subcore_axis_name="s", num_cores=NC, num_subcores=NS
)


@functools.partial(
    pl.kernel,
    out_type=jax.ShapeDtypeStruct((BATCH, EMBED), jnp.float32),
    mesh=_mesh,
    scratch_types=[
        pltpu.VMEM((IDX_PER_W,), jnp.int32),
        pltpu.VMEM((2, CHUNK_IDX, EMBED), jnp.float32),
        pltpu.VMEM((N_CHUNKS, CHUNK_OBJ, EMBED), jnp.float32),
        pltpu.SemaphoreType.DMA,
        pltpu.SemaphoreType.DMA,
    ],
)
def _pooled_lookup(idx_hbm, tbl_hbm, out_hbm, idx_v, rows_v, out_v, gsem, osem):
    wid = lax.axis_index("s") * NC + lax.axis_index("c")
    ibase = wid * IDX_PER_W
    pltpu.sync_copy(idx_hbm.at[pl.ds(ibase, IDX_PER_W)], idx_v)

    def fire(j, buf):
        for g in range(GRPS):
            pltpu.async_copy(
                tbl_hbm.at[idx_v.at[pl.ds((j * GRPS + g) * IDX_GRP, IDX_GRP)]],
                rows_v.at[buf].at[pl.ds(g * IDX_GRP, IDX_GRP)],
                gsem,
            )

    def wait_gathers():
        for _ in range(GRPS):
            pltpu.make_async_copy(
                tbl_hbm.at[pl.ds(0, IDX_GRP)],
                rows_v.at[0].at[pl.ds(0, IDX_GRP)],
                gsem,
            ).wait()

    def pool(j, buf):
        def _pool(cc, _):
            for u in range(2):
                c = cc * 2 + u
                r0 = c * WORDS
                for d in range(EMBED // LANES):
                    sl = pl.ds(d * LANES, LANES)
                    vals = [rows_v[buf, r0 + l, sl] for l in range(WORDS)]
                    while len(vals) > 1:
                        vals = [
                            vals[i] + vals[i + 1]
                            for i in range(0, len(vals) - 1, 2)
                        ] + ([vals[-1]] if len(vals) % 2 else [])
                    out_v[j, c, sl] = vals[0] * jnp.float32(1.0 / WORDS)
            return 0

        lax.fori_loop(0, CHUNK_OBJ // 2, _pool, 0)
        pltpu.async_copy(
            out_v.at[j],
            out_hbm.at[pl.ds(wid * OBJ_PER_W + j * CHUNK_OBJ, CHUNK_OBJ)],
            osem,
        )

    fire(0, 0)

    def step(j, _):
        buf = lax.rem(j, 2)
        wait_gathers()
        fire(j + 1, 1 - buf)
        pool(j, buf)
        return 0

    lax.fori_loop(0, N_CHUNKS - 1, step, 0)
    wait_gathers()
    pool(N_CHUNKS - 1, (N_CHUNKS - 1) % 2)
    for k in range(N_CHUNKS):
        pltpu.make_async_copy(
            out_v.at[k], out_hbm.at[pl.ds(0, CHUNK_OBJ)], osem
        ).wait()


def kernel(indices, vectors):
    idx = indices.astype(jnp.int32).reshape(BATCH * WORDS)
    return _pooled_lookup(idx, vectors)


# speedup vs baseline: 1.0148x; 1.0148x over previous
import functools

import jax
import jax.numpy as jnp
from jax import lax
from jax.experimental import pallas as pl
from jax.experimental.pallas import tpu as pltpu
from jax.experimental.pallas import tpu_sc as plsc

BATCH = 4096
WORDS = 20
EMBED = 128
LANES = 16
NC, NS = 2, 16
NW = NC * NS
OBJ_PER_W = BATCH // NW
CHUNK_OBJ = 16
CHUNK_IDX = CHUNK_OBJ * WORDS
IDX_GRP = 80
GRPS = CHUNK_IDX // IDX_GRP
N_CHUNKS = OBJ_PER_W // CHUNK_OBJ
IDX_PER_W = OBJ_PER_W * WORDS
ROW_BYTES = EMBED * 4

_mesh = plsc.VectorSubcoreMesh(
    core_axis_name="c", subcore_axis_name="s", num_cores=NC, num_subcores=NS
)


@functools.partial(
    pl.kernel,
    out_type=jax.ShapeDtypeStruct((BATCH, EMBED), jnp.float32),
    mesh=_mesh,
    scratch_types=[
        pltpu.VMEM((IDX_PER_W,), jnp.int32),
        pltpu.VMEM((2, CHUNK_IDX, EMBED), jnp.float32),
        pltpu.VMEM((N_CHUNKS, CHUNK_OBJ, EMBED), jnp.float32),
        pltpu.SemaphoreType.DMA,
        pltpu.SemaphoreType.DMA,
    ],
)
def _pooled_lookup(idx_hbm, tbl_hbm, out_hbm, idx_v, rows_v, out_v, gsem, osem):
    wid = lax.axis_index("s") * NC + lax.axis_index("c")
    ibase = wid * IDX_PER_W
    pltpu.sync_copy(idx_hbm.at[pl.ds(ibase, IDX_PER_W)], idx_v)

    def fire(j, buf):
        for g in range(GRPS):
            pltpu.async_copy(
                tbl_hbm.at[idx_v.at[pl.ds((j * GRPS + g) * IDX_GRP, IDX_GRP)]],
                rows_v.at[buf].at[pl.ds(g * IDX_GRP, IDX_GRP)],
                gsem,
            )

    def wait_gathers():
        for _ in range(GRPS):
            pltpu.make_async_copy(
                tbl_hbm.at[pl.ds(0, IDX_GRP)],
                rows_v.at[0].at[pl.ds(0, IDX_GRP)],
                gsem,
            ).wait()

    def pool(j, buf):
        def _pool(c, _):
            r0 = c * WORDS
            for d in range(EMBED // LANES):
                sl = pl.ds(d * LANES, LANES)
                vals = [rows_v[buf, r0 + l, sl] for l in range(WORDS)]
                while len(vals) > 1:
                    vals = [
                        vals[i] + vals[i + 1] for i in range(0, len(vals) - 1, 2)
                    ] + ([vals[-1]] if len(vals) % 2 else [])
                out_v[j, c, sl] = vals[0] * jnp.float32(1.0 / WORDS)
            return 0

        lax.fori_loop(0, CHUNK_OBJ, _pool, 0)
        pltpu.async_copy(
            out_v.at[j],
            out_hbm.at[pl.ds(wid * OBJ_PER_W + j * CHUNK_OBJ, CHUNK_OBJ)],
            osem,
        )

    fire(0, 0)

    def step(j, _):
        buf = lax.rem(j, 2)
        wait_gathers()
        fire(j + 1, 1 - buf)
        pool(j, buf)
        return 0

    lax.fori_loop(0, N_CHUNKS - 1, step, 0)
    wait_gathers()
    pool(N_CHUNKS - 1, (N_CHUNKS - 1) % 2)
    for k in range(N_CHUNKS):
        pltpu.make_async_copy(
            out_v.at[k], out_hbm.at[pl.ds(0, CHUNK_OBJ)], osem
        ).wait()


def kernel(indices, vectors):
    idx = indices.astype(jnp.int32).reshape(BATCH * WORDS)
    return _pooled_lookup(idx, vectors)
